# baseline (device time: 59932 ns/iter reference)
import jax
import jax.numpy as jnp
from jax import lax
from jax.experimental import pallas as pl
from jax.experimental.pallas import tpu as pltpu

N_DEV = 8
B = 2
SQ = 512
SKV_LOC = 512
HQ = 8
DH = 64
DM = 768
DQ = HQ * DH
CHUNK = SQ // N_DEV
NEG = -1e9


def _expand(s, rows):
    return jnp.concatenate(
        [jnp.broadcast_to(s[:, h : h + 1], (rows, DH)) for h in range(HQ)],
        axis=1,
    )


def _combine(acc_s, rec_s, acc_o, rec_o, rows):
    new_o = []
    cols = []
    for b in range(B):
        am = acc_s[:, b * 16 : b * 16 + 8]
        al = acc_s[:, b * 16 + 8 : b * 16 + 16]
        rm = rec_s[:, b * 16 : b * 16 + 8]
        rl = rec_s[:, b * 16 + 8 : b * 16 + 16]
        nm = jnp.maximum(am, rm)
        sa = jnp.exp(am - nm)
        sb = jnp.exp(rm - nm)
        nl = al * sa + rl * sb
        new_o.append(acc_o[b] * _expand(sa, rows) + rec_o[b] * _expand(sb, rows))
        cols += [nm, nl]
    return jnp.concatenate(cols, axis=1), new_o


def kernel(x, Wq, K_ext, V_ext, Wo):
    def body(
        x_ref,
        wq_ref,
        k_ref,
        v_ref,
        wo_ref,
        out_ref,
        o_work,
        s_work,
        recv_o,
        recv_s,
        ctx_ref,
        sem_rs_send,
        sem_rs_recv_o,
        sem_rs_recv_s,
        sem_rs_send_s,
        sem_ag_send,
        sem_ag_recv,
    ):
        my = lax.axis_index("i")

        barrier = pltpu.get_barrier_semaphore()
        for k in range(1, N_DEV):
            pl.semaphore_signal(
                barrier,
                inc=1,
                device_id=(lax.rem(my + k, N_DEV),),
                device_id_type=pl.DeviceIdType.MESH,
            )
        pl.semaphore_wait(barrier, N_DEV - 1)

        kv0 = my * SKV_LOC
        qb = lax.broadcasted_iota(jnp.int32, (SQ, SKV_LOC), 0) // 64
        kb = (kv0 + lax.broadcasted_iota(jnp.int32, (SQ, SKV_LOC), 1)) // 64
        mask = (qb == kb) | (kb == 0) | ((qb + kb) % 3 == 0)

        wq_bf = wq_ref[...].astype(jnp.bfloat16)
        s_cols = []
        for b in range(B):
            q_b = jnp.dot(
                x_ref[b].astype(jnp.bfloat16),
                wq_bf,
                preferred_element_type=jnp.float32,
            ).astype(jnp.bfloat16)
            m_cols = []
            l_cols = []
            o_blocks = []
            for h in range(HQ):
                q_bh = q_b[:, h * DH : (h + 1) * DH]
                k_bh = k_ref[b, :, h, :].astype(jnp.bfloat16)
                s = (
                    lax.dot_general(
                        q_bh,
                        k_bh,
                        (((1,), (1,)), ((), ())),
                        preferred_element_type=jnp.float32,
                    )
                    * 0.125
                )
                s = jnp.where(mask, s, NEG)
                m_bh = jnp.max(s, axis=1, keepdims=True)
                w = jnp.exp(s - m_bh)
                l_bh = jnp.sum(w, axis=1, keepdims=True)
                o_blocks.append(
                    jnp.dot(
                        w.astype(jnp.bfloat16),
                        v_ref[b, :, h, :].astype(jnp.bfloat16),
                        preferred_element_type=jnp.float32,
                    )
                )
                m_cols.append(m_bh)
                l_cols.append(l_bh)
            o_b = jnp.concatenate(o_blocks, axis=1)
            o_work[b] = o_b.astype(jnp.bfloat16)
            s_cols += m_cols + l_cols
        s_all = jnp.concatenate(s_cols, axis=1)
        s_work[...] = s_all

        rs_rdmas = []
        for k in range(1, N_DEV):
            p = lax.rem(my + k, N_DEV)
            slot = N_DEV - 1 - k
            rdma_o = pltpu.make_async_remote_copy(
                src_ref=o_work.at[:, pl.ds(p * CHUNK, CHUNK), :],
                dst_ref=recv_o.at[slot],
                send_sem=sem_rs_send.at[k - 1],
                recv_sem=sem_rs_recv_o.at[slot],
                device_id=(p,),
                device_id_type=pl.DeviceIdType.MESH,
            )
            rdma_s = pltpu.make_async_remote_copy(
                src_ref=s_work.at[pl.ds(p * CHUNK, CHUNK), :],
                dst_ref=recv_s.at[slot],
                send_sem=sem_rs_send_s.at[k - 1],
                recv_sem=sem_rs_recv_s.at[slot],
                device_id=(p,),
                device_id_type=pl.DeviceIdType.MESH,
            )
            rdma_o.start()
            rdma_s.start()
            rs_rdmas.append((rdma_o, rdma_s))

        acc_o = [
            o_work[b, pl.ds(my * CHUNK, CHUNK), :].astype(jnp.float32)
            for b in range(B)
        ]
        acc_s = s_work[pl.ds(my * CHUNK, CHUNK), :]

        for idx, (rdma_o, rdma_s) in enumerate(rs_rdmas):
            rdma_o.wait()
            rdma_s.wait()
            slot = N_DEV - 2 - idx
            rec_o = [recv_o[slot, b].astype(jnp.float32) for b in range(B)]
            acc_s, acc_o = _combine(acc_s, recv_s[slot], acc_o, rec_o, CHUNK)

        ctx_own = []
        for b in range(B):
            l_b = acc_s[:, b * 16 + 8 : b * 16 + 16]
            c_b = acc_o[b] / _expand(l_b, CHUNK)
            ctx_own.append(c_b)
            ctx_ref[b, pl.ds(my * CHUNK, CHUNK), :] = c_b.astype(jnp.bfloat16)

        ag_rdmas = []
        for k in range(1, N_DEV):
            p = lax.rem(my + k, N_DEV)
            rdma = pltpu.make_async_remote_copy(
                src_ref=ctx_ref.at[:, pl.ds(my * CHUNK, CHUNK), :],
                dst_ref=ctx_ref.at[:, pl.ds(my * CHUNK, CHUNK), :],
                send_sem=sem_ag_send.at[k - 1],
                recv_sem=sem_ag_recv.at[N_DEV - 1 - k],
                device_id=(p,),
                device_id_type=pl.DeviceIdType.MESH,
            )
            rdma.start()
            ag_rdmas.append(rdma)

        wo_bf = wo_ref[...].astype(jnp.bfloat16)

        for b in range(B):
            out_ref[b, pl.ds(my * CHUNK, CHUNK), :] = jnp.dot(
                ctx_own[b].astype(jnp.bfloat16),
                wo_bf,
                preferred_element_type=jnp.float32,
            )

        for k, rdma in enumerate(ag_rdmas, start=1):
            rdma.wait()
            src = lax.rem(my - k + N_DEV, N_DEV)
            for b in range(B):
                blk = ctx_ref[b, pl.ds(src * CHUNK, CHUNK), :]
                out_ref[b, pl.ds(src * CHUNK, CHUNK), :] = jnp.dot(
                    blk, wo_bf, preferred_element_type=jnp.float32
                )

    return pl.pallas_call(
        body,
        out_shape=jax.ShapeDtypeStruct((B, SQ, DM), jnp.float32),
        in_specs=[pl.BlockSpec(memory_space=pltpu.VMEM)] * 5,
        out_specs=pl.BlockSpec(memory_space=pltpu.VMEM),
        scratch_shapes=[
            pltpu.VMEM((B, SQ, DQ), jnp.bfloat16),
            pltpu.VMEM((SQ, 32), jnp.float32),
            pltpu.VMEM((N_DEV - 1, B, CHUNK, DQ), jnp.bfloat16),
            pltpu.VMEM((N_DEV - 1, CHUNK, 32), jnp.float32),
            pltpu.VMEM((B, SQ, DQ), jnp.bfloat16),
            pltpu.SemaphoreType.DMA((N_DEV - 1,)),
            pltpu.SemaphoreType.DMA((N_DEV - 1,)),
            pltpu.SemaphoreType.DMA((N_DEV - 1,)),
            pltpu.SemaphoreType.DMA((N_DEV - 1,)),
            pltpu.SemaphoreType.DMA((N_DEV - 1,)),
            pltpu.SemaphoreType.DMA((N_DEV - 1,)),
        ],
        compiler_params=pltpu.CompilerParams(
            collective_id=0, vmem_limit_bytes=100 * 1024 * 1024
        ),
    )(x, Wq, K_ext, V_ext, Wo)


# device time: 53847 ns/iter; 1.1130x vs baseline; 1.1130x over previous
import jax
import jax.numpy as jnp
from jax import lax
from jax.experimental import pallas as pl
from jax.experimental.pallas import tpu as pltpu

N_DEV = 8
B = 2
SQ = 512
SKV_LOC = 512
HQ = 8
DH = 64
DM = 768
DQ = HQ * DH
CHUNK = SQ // N_DEV
NG = 2
HG = HQ // NG
GD = HG * DH
NEG = -1e9


def _expand(s, rows, hq):
    return jnp.concatenate(
        [jnp.broadcast_to(s[:, h : h + 1], (rows, DH)) for h in range(hq)],
        axis=1,
    )


def _combine(acc_s, rec_s, acc_o, rec_o, rows, hq):
    new_o = []
    cols = []
    for b in range(B):
        base = b * 2 * hq
        am = acc_s[:, base : base + hq]
        al = acc_s[:, base + hq : base + 2 * hq]
        rm = rec_s[:, base : base + hq]
        rl = rec_s[:, base + hq : base + 2 * hq]
        nm = jnp.maximum(am, rm)
        sa = jnp.exp(am - nm)
        sb = jnp.exp(rm - nm)
        nl = al * sa + rl * sb
        new_o.append(
            acc_o[b] * _expand(sa, rows, hq) + rec_o[b] * _expand(sb, rows, hq)
        )
        cols += [nm, nl]
    return jnp.concatenate(cols, axis=1), new_o


def kernel(x, Wq, K_ext, V_ext, Wo):
    def body(
        x_ref,
        wq_ref,
        k_ref,
        v_ref,
        wo_ref,
        out_ref,
        o_work_a,
        o_work_b,
        s_work_a,
        s_work_b,
        recv_o_a,
        recv_o_b,
        recv_s_a,
        recv_s_b,
        ctx_ref,
        sem_rs_send_o,
        sem_rs_recv_o,
        sem_rs_send_s,
        sem_rs_recv_s,
        sem_ag_send,
        sem_ag_recv,
    ):
        my = lax.axis_index("i")
        o_work = (o_work_a, o_work_b)
        s_work = (s_work_a, s_work_b)
        recv_o = (recv_o_a, recv_o_b)
        recv_s = (recv_s_a, recv_s_b)

        barrier = pltpu.get_barrier_semaphore()
        for k in range(1, N_DEV):
            pl.semaphore_signal(
                barrier,
                inc=1,
                device_id=(lax.rem(my + k, N_DEV),),
                device_id_type=pl.DeviceIdType.MESH,
            )
        pl.semaphore_wait(barrier, N_DEV - 1)

        kv0 = my * SKV_LOC
        qb = lax.broadcasted_iota(jnp.int32, (SQ, SKV_LOC), 0) // 64
        kb = (kv0 + lax.broadcasted_iota(jnp.int32, (SQ, SKV_LOC), 1)) // 64
        mask = (qb == kb) | (kb == 0) | ((qb + kb) % 3 == 0)

        q = [
            jnp.dot(x_ref[b], wq_ref[...], preferred_element_type=jnp.float32)
            for b in range(B)
        ]

        rs_rdmas = [[], []]
        for g in range(NG):
            s_cols = []
            for b in range(B):
                m_cols = []
                l_cols = []
                o_blocks = []
                for h in range(g * HG, (g + 1) * HG):
                    q_bh = q[b][:, h * DH : (h + 1) * DH]
                    k_bh = k_ref[b, :, h, :]
                    s = (
                        lax.dot_general(
                            q_bh,
                            k_bh,
                            (((1,), (1,)), ((), ())),
                            preferred_element_type=jnp.float32,
                        )
                        * 0.125
                    )
                    s = jnp.where(mask, s, NEG)
                    m_bh = jnp.max(s, axis=1, keepdims=True)
                    w = jnp.exp(s - m_bh)
                    l_bh = jnp.sum(w, axis=1, keepdims=True)
                    o_blocks.append(
                        jnp.dot(
                            w,
                            v_ref[b, :, h, :],
                            preferred_element_type=jnp.float32,
                        )
                    )
                    m_cols.append(m_bh)
                    l_cols.append(l_bh)
                o_work[g][b] = jnp.concatenate(o_blocks, axis=1).astype(
                    jnp.bfloat16
                )
                s_cols += m_cols + l_cols
            s_work[g][...] = jnp.concatenate(s_cols, axis=1)

            for k in range(1, N_DEV):
                p = lax.rem(my + k, N_DEV)
                slot = N_DEV - 1 - k
                rdma_o = pltpu.make_async_remote_copy(
                    src_ref=o_work[g].at[:, pl.ds(p * CHUNK, CHUNK), :],
                    dst_ref=recv_o[g].at[slot],
                    send_sem=sem_rs_send_o.at[g, k - 1],
                    recv_sem=sem_rs_recv_o.at[g, slot],
                    device_id=(p,),
                    device_id_type=pl.DeviceIdType.MESH,
                )
                rdma_s = pltpu.make_async_remote_copy(
                    src_ref=s_work[g].at[pl.ds(p * CHUNK, CHUNK), :],
                    dst_ref=recv_s[g].at[slot],
                    send_sem=sem_rs_send_s.at[g, k - 1],
                    recv_sem=sem_rs_recv_s.at[g, slot],
                    device_id=(p,),
                    device_id_type=pl.DeviceIdType.MESH,
                )
                rdma_o.start()
                rdma_s.start()
                rs_rdmas[g].append((rdma_o, rdma_s))

        ctx_own = [[None, None] for _ in range(B)]
        for g in range(NG):
            acc_o = [
                o_work[g][b, pl.ds(my * CHUNK, CHUNK), :].astype(jnp.float32)
                for b in range(B)
            ]
            acc_s = s_work[g][pl.ds(my * CHUNK, CHUNK), :]
            for idx, (rdma_o, rdma_s) in enumerate(rs_rdmas[g]):
                rdma_o.wait()
                rdma_s.wait()
                slot = N_DEV - 2 - idx
                rec_o = [
                    recv_o[g][slot, b].astype(jnp.float32) for b in range(B)
                ]
                acc_s, acc_o = _combine(
                    acc_s, recv_s[g][slot], acc_o, rec_o, CHUNK, HG
                )
            for b in range(B):
                l_b = acc_s[:, b * 2 * HG + HG : b * 2 * HG + 2 * HG]
                c_b = acc_o[b] / _expand(l_b, CHUNK, HG)
                ctx_own[b][g] = c_b
                ctx_ref[
                    b, pl.ds(my * CHUNK, CHUNK), g * GD : (g + 1) * GD
                ] = c_b.astype(jnp.bfloat16)

        ag_rdmas = []
        for k in range(1, N_DEV):
            p = lax.rem(my + k, N_DEV)
            rdma = pltpu.make_async_remote_copy(
                src_ref=ctx_ref.at[:, pl.ds(my * CHUNK, CHUNK), :],
                dst_ref=ctx_ref.at[:, pl.ds(my * CHUNK, CHUNK), :],
                send_sem=sem_ag_send.at[k - 1],
                recv_sem=sem_ag_recv.at[N_DEV - 1 - k],
                device_id=(p,),
                device_id_type=pl.DeviceIdType.MESH,
            )
            rdma.start()
            ag_rdmas.append(rdma)

        wo_bf = wo_ref[...].astype(jnp.bfloat16)

        for b in range(B):
            own = jnp.concatenate(ctx_own[b], axis=1)
            out_ref[b, pl.ds(my * CHUNK, CHUNK), :] = jnp.dot(
                own.astype(jnp.bfloat16),
                wo_bf,
                preferred_element_type=jnp.float32,
            )

        for k, rdma in enumerate(ag_rdmas, start=1):
            rdma.wait()
            src = lax.rem(my - k + N_DEV, N_DEV)
            for b in range(B):
                blk = ctx_ref[b, pl.ds(src * CHUNK, CHUNK), :]
                out_ref[b, pl.ds(src * CHUNK, CHUNK), :] = jnp.dot(
                    blk, wo_bf, preferred_element_type=jnp.float32
                )

    return pl.pallas_call(
        body,
        out_shape=jax.ShapeDtypeStruct((B, SQ, DM), jnp.float32),
        in_specs=[pl.BlockSpec(memory_space=pltpu.VMEM)] * 5,
        out_specs=pl.BlockSpec(memory_space=pltpu.VMEM),
        scratch_shapes=[
            pltpu.VMEM((B, SQ, GD), jnp.bfloat16),
            pltpu.VMEM((B, SQ, GD), jnp.bfloat16),
            pltpu.VMEM((SQ, 16), jnp.float32),
            pltpu.VMEM((SQ, 16), jnp.float32),
            pltpu.VMEM((N_DEV - 1, B, CHUNK, GD), jnp.bfloat16),
            pltpu.VMEM((N_DEV - 1, B, CHUNK, GD), jnp.bfloat16),
            pltpu.VMEM((N_DEV - 1, CHUNK, 16), jnp.float32),
            pltpu.VMEM((N_DEV - 1, CHUNK, 16), jnp.float32),
            pltpu.VMEM((B, SQ, DQ), jnp.bfloat16),
            pltpu.SemaphoreType.DMA((NG, N_DEV - 1)),
            pltpu.SemaphoreType.DMA((NG, N_DEV - 1)),
            pltpu.SemaphoreType.DMA((NG, N_DEV - 1)),
            pltpu.SemaphoreType.DMA((NG, N_DEV - 1)),
            pltpu.SemaphoreType.DMA((N_DEV - 1,)),
            pltpu.SemaphoreType.DMA((N_DEV - 1,)),
        ],
        compiler_params=pltpu.CompilerParams(
            collective_id=0, vmem_limit_bytes=100 * 1024 * 1024
        ),
    )(x, Wq, K_ext, V_ext, Wo)


# device time: 51651 ns/iter; 1.1603x vs baseline; 1.0425x over previous
import jax
import jax.numpy as jnp
from jax import lax
from jax.experimental import pallas as pl
from jax.experimental.pallas import tpu as pltpu

N_DEV = 8
B = 2
SQ = 512
SKV_LOC = 512
HQ = 8
DH = 64
DM = 768
DQ = HQ * DH
CHUNK = SQ // N_DEV
NG = 2
HG = HQ // NG
GD = HG * DH
SW = 2 * B * HG
PACK = 128 // SW
NEG = -1e9


def _expand(s, rows, hq):
    return jnp.concatenate(
        [jnp.broadcast_to(s[:, h : h + 1], (rows, DH)) for h in range(hq)],
        axis=1,
    )


def _combine(acc_s, rec_s, acc_o, rec_o, rows, hq):
    new_o = []
    cols = []
    for b in range(B):
        base = b * 2 * hq
        am = acc_s[:, base : base + hq]
        al = acc_s[:, base + hq : base + 2 * hq]
        rm = rec_s[:, base : base + hq]
        rl = rec_s[:, base + hq : base + 2 * hq]
        nm = jnp.maximum(am, rm)
        sa = jnp.exp(am - nm)
        sb = jnp.exp(rm - nm)
        nl = al * sa + rl * sb
        new_o.append(
            acc_o[b] * _expand(sa, rows, hq) + rec_o[b] * _expand(sb, rows, hq)
        )
        cols += [nm, nl]
    return jnp.concatenate(cols, axis=1), new_o


def kernel(x, Wq, K_ext, V_ext, Wo):
    def body(
        x_ref,
        wq_ref,
        k_ref,
        v_ref,
        wo_ref,
        out_ref,
        o_work_a,
        o_work_b,
        s_work_a,
        s_work_b,
        recv_o_a,
        recv_o_b,
        recv_s_a,
        recv_s_b,
        ctx_ref,
        sem_rs_send_o,
        sem_rs_recv_o,
        sem_rs_send_s,
        sem_rs_recv_s,
        sem_ag_send,
        sem_ag_recv,
    ):
        my = lax.axis_index("i")
        o_work = (o_work_a, o_work_b)
        s_work = (s_work_a, s_work_b)
        recv_o = (recv_o_a, recv_o_b)
        recv_s = (recv_s_a, recv_s_b)

        barrier = pltpu.get_barrier_semaphore()
        for k in range(1, N_DEV):
            pl.semaphore_signal(
                barrier,
                inc=1,
                device_id=(lax.rem(my + k, N_DEV),),
                device_id_type=pl.DeviceIdType.MESH,
            )
        pl.semaphore_wait(barrier, N_DEV - 1)

        kv0 = my * SKV_LOC
        qb = lax.broadcasted_iota(jnp.int32, (SQ, SKV_LOC), 0) // 64
        kb = (kv0 + lax.broadcasted_iota(jnp.int32, (SQ, SKV_LOC), 1)) // 64
        mask = (qb == kb) | (kb == 0) | ((qb + kb) % 3 == 0)

        q = [
            jnp.dot(x_ref[b], wq_ref[...], preferred_element_type=jnp.float32)
            for b in range(B)
        ]

        rs_rdmas = [[], []]
        for g in range(NG):
            s_cols = []
            for b in range(B):
                m_cols = []
                l_cols = []
                o_blocks = []
                for h in range(g * HG, (g + 1) * HG):
                    q_bh = q[b][:, h * DH : (h + 1) * DH]
                    k_bh = k_ref[b, :, h, :]
                    s = (
                        lax.dot_general(
                            q_bh,
                            k_bh,
                            (((1,), (1,)), ((), ())),
                            preferred_element_type=jnp.float32,
                        )
                        * 0.125
                    )
                    s = jnp.where(mask, s, NEG)
                    m_bh = jnp.max(s, axis=1, keepdims=True)
                    w = jnp.exp(s - m_bh)
                    l_bh = jnp.sum(w, axis=1, keepdims=True)
                    o_blocks.append(
                        jnp.dot(
                            w,
                            v_ref[b, :, h, :],
                            preferred_element_type=jnp.float32,
                        )
                    )
                    m_cols.append(m_bh)
                    l_cols.append(l_bh)
                o_work[g][b] = jnp.concatenate(o_blocks, axis=1).astype(
                    jnp.bfloat16
                )
                s_cols += m_cols + l_cols
            s_work[g][...] = jnp.concatenate(s_cols, axis=1).astype(
                jnp.bfloat16
            )

            for k in range(1, N_DEV):
                p = lax.rem(my + k, N_DEV)
                slot = N_DEV - 1 - k
                rdma_o = pltpu.make_async_remote_copy(
                    src_ref=o_work[g].at[:, pl.ds(p * CHUNK, CHUNK), :],
                    dst_ref=recv_o[g].at[slot],
                    send_sem=sem_rs_send_o.at[g, k - 1],
                    recv_sem=sem_rs_recv_o.at[g, slot],
                    device_id=(p,),
                    device_id_type=pl.DeviceIdType.MESH,
                )
                rdma_s = pltpu.make_async_remote_copy(
                    src_ref=s_work[g].at[pl.ds(p * CHUNK, CHUNK), :],
                    dst_ref=recv_s[g].at[slot],
                    send_sem=sem_rs_send_s.at[g, k - 1],
                    recv_sem=sem_rs_recv_s.at[g, slot],
                    device_id=(p,),
                    device_id_type=pl.DeviceIdType.MESH,
                )
                rdma_o.start()
                rdma_s.start()
                rs_rdmas[g].append((rdma_o, rdma_s))

        ctx_own = [[None, None] for _ in range(B)]
        ag_rdmas = [[], []]
        for g in range(NG):
            acc_o = [
                o_work[g][b, pl.ds(my * CHUNK, CHUNK), :].astype(jnp.float32)
                for b in range(B)
            ]
            acc_s = s_work[g][pl.ds(my * CHUNK, CHUNK), :].astype(jnp.float32)
            for idx, (rdma_o, rdma_s) in enumerate(rs_rdmas[g]):
                rdma_o.wait()
                rdma_s.wait()
                slot = N_DEV - 2 - idx
                rec_o = [
                    recv_o[g][slot, b].astype(jnp.float32) for b in range(B)
                ]
                rec_s = recv_s[g][slot].astype(jnp.float32)
                acc_s, acc_o = _combine(acc_s, rec_s, acc_o, rec_o, CHUNK, HG)
            for b in range(B):
                l_b = acc_s[:, b * 2 * HG + HG : b * 2 * HG + 2 * HG]
                c_b = acc_o[b] / _expand(l_b, CHUNK, HG)
                ctx_own[b][g] = c_b
                ctx_ref[
                    b, pl.ds(my * CHUNK, CHUNK), g * GD : (g + 1) * GD
                ] = c_b.astype(jnp.bfloat16)
            for k in range(1, N_DEV):
                p = lax.rem(my + k, N_DEV)
                rdma = pltpu.make_async_remote_copy(
                    src_ref=ctx_ref.at[
                        :, pl.ds(my * CHUNK, CHUNK), g * GD : (g + 1) * GD
                    ],
                    dst_ref=ctx_ref.at[
                        :, pl.ds(my * CHUNK, CHUNK), g * GD : (g + 1) * GD
                    ],
                    send_sem=sem_ag_send.at[g, k - 1],
                    recv_sem=sem_ag_recv.at[g, N_DEV - 1 - k],
                    device_id=(p,),
                    device_id_type=pl.DeviceIdType.MESH,
                )
                rdma.start()
                ag_rdmas[g].append(rdma)

        wo_bf = wo_ref[...].astype(jnp.bfloat16)

        for b in range(B):
            own = jnp.concatenate(ctx_own[b], axis=1)
            out_ref[b, pl.ds(my * CHUNK, CHUNK), :] = jnp.dot(
                own.astype(jnp.bfloat16),
                wo_bf,
                preferred_element_type=jnp.float32,
            )

        for k in range(1, N_DEV):
            ag_rdmas[0][k - 1].wait()
            ag_rdmas[1][k - 1].wait()
            src = lax.rem(my - k + N_DEV, N_DEV)
            for b in range(B):
                blk = ctx_ref[b, pl.ds(src * CHUNK, CHUNK), :]
                out_ref[b, pl.ds(src * CHUNK, CHUNK), :] = jnp.dot(
                    blk, wo_bf, preferred_element_type=jnp.float32
                )

    return pl.pallas_call(
        body,
        out_shape=jax.ShapeDtypeStruct((B, SQ, DM), jnp.float32),
        in_specs=[pl.BlockSpec(memory_space=pltpu.VMEM)] * 5,
        out_specs=pl.BlockSpec(memory_space=pltpu.VMEM),
        scratch_shapes=[
            pltpu.VMEM((B, SQ, GD), jnp.bfloat16),
            pltpu.VMEM((B, SQ, GD), jnp.bfloat16),
            pltpu.VMEM((SQ, SW), jnp.bfloat16),
            pltpu.VMEM((SQ, SW), jnp.bfloat16),
            pltpu.VMEM((N_DEV - 1, B, CHUNK, GD), jnp.bfloat16),
            pltpu.VMEM((N_DEV - 1, B, CHUNK, GD), jnp.bfloat16),
            pltpu.VMEM((N_DEV - 1, CHUNK, SW), jnp.bfloat16),
            pltpu.VMEM((N_DEV - 1, CHUNK, SW), jnp.bfloat16),
            pltpu.VMEM((B, SQ, DQ), jnp.bfloat16),
            pltpu.SemaphoreType.DMA((NG, N_DEV - 1)),
            pltpu.SemaphoreType.DMA((NG, N_DEV - 1)),
            pltpu.SemaphoreType.DMA((NG, N_DEV - 1)),
            pltpu.SemaphoreType.DMA((NG, N_DEV - 1)),
            pltpu.SemaphoreType.DMA((NG, N_DEV - 1)),
            pltpu.SemaphoreType.DMA((NG, N_DEV - 1)),
        ],
        compiler_params=pltpu.CompilerParams(
            collective_id=0, vmem_limit_bytes=100 * 1024 * 1024
        ),
    )(x, Wq, K_ext, V_ext, Wo)
